# baseline (speedup 1.0000x reference)
import functools

import jax
import jax.numpy as jnp
from jax import lax
from jax.experimental import pallas as pl
from jax.experimental.pallas import tpu as pltpu
from jax.experimental.pallas import tpu_sc as plsc

HASH_BUCKET_SIZE = 32768
GEO_BUCKETS = 4096
NCAT = 26
CAT_DIM = 16
REGION_EMB_DIM = 32
HIST = 200
NUM = 64
K = 64
TOPK = 8
HIDDEN = 256

_SQRT_HALF = 0.7071067811865476

_NC = 2
_NS = 16
_NW = _NC * _NS
_CHUNK = 128
_XW = 512
_CAT_COLS = NCAT * CAT_DIM
_FEAT = _CAT_COLS + REGION_EMB_DIM


def _sc_gather_call(tbl128, idx_fmajor, rtbl128, region_id):
    n_idx = idx_fmajor.shape[0]
    n_reg = region_id.shape[0]
    per_w = n_idx // _NW
    rows_per_blk = _CHUNK * NCAT
    blk_per_w = per_w // rows_per_blk
    assert per_w % rows_per_blk == 0 and n_reg % (_NW * _CHUNK) == 0

    mesh = plsc.VectorSubcoreMesh(core_axis_name="c", subcore_axis_name="s")

    @functools.partial(
        pl.kernel,
        mesh=mesh,
        compiler_params=pltpu.CompilerParams(use_tc_tiling_on_sc=True,
                                             needs_layout_passes=False),
        out_type=jax.ShapeDtypeStruct((n_reg, _XW), jnp.float32),
        scratch_types=[
            pltpu.VMEM((rows_per_blk,), jnp.int32),
            pltpu.VMEM((rows_per_blk,), jnp.int32),
            pltpu.VMEM((rows_per_blk,), jnp.int32),
            pltpu.VMEM((_CHUNK, 128), jnp.float32),
            pltpu.VMEM((_CHUNK, 128), jnp.float32),
            pltpu.VMEM((_CHUNK, _XW), jnp.float32),
            pltpu.VMEM((_CHUNK,), jnp.int32),
            pltpu.VMEM((_CHUNK,), jnp.int32),
            pltpu.VMEM((_CHUNK,), jnp.int32),
            pltpu.SemaphoreType.DMA,
            pltpu.SemaphoreType.DMA,
            pltpu.SemaphoreType.DMA,
        ],
    )
    def sc_kernel(tbl_hbm, idx_hbm, rtbl_hbm, ridx_hbm, x_out,
                  raw_v, t_v, o_v, g0, g1, asm_v, rraw_v, rt_v, ro_v,
                  s0, s1, so):
        w = lax.axis_index("s") * _NC + lax.axis_index("c")

        def block_body(o, carry):
            blk = w * blk_per_w + o

            def fire(buf, sem, f):
                return pltpu.async_copy(
                    tbl_hbm.at[t_v.at[pl.ds(f * _CHUNK, _CHUNK)]], buf, sem)

            def extract(buf, f):
                def step(p, carry2):
                    row_vec = p * 16 + lax.iota(jnp.int32, 16)
                    off_vec = o_v[pl.ds(f * _CHUNK + p * 16, 16)]
                    for d in range(CAT_DIM):
                        val = plsc.load_gather(buf, [row_vec, off_vec + d])
                        colv = lax.broadcast(f * CAT_DIM + d, (16,))
                        plsc.store_scatter(asm_v, [row_vec, colv], val)
                    return carry2
                lax.fori_loop(0, _CHUNK // 16, step, 0)

            def flush(cb):
                return pltpu.async_copy(
                    asm_v.at[pl.ds(0, _CHUNK), pl.ds(cb * 128, 128)],
                    x_out.at[pl.ds(blk * _CHUNK, _CHUNK),
                             pl.ds(cb * 128, 128)],
                    so)

            pltpu.sync_copy(idx_hbm.at[pl.ds(blk * rows_per_blk,
                                             rows_per_blk)], raw_v)

            def fix(i, c):
                posl = i * 16 + lax.iota(jnp.int32, 16)
                fld = lax.shift_right_logical(posl, 7)
                raw = raw_v[pl.ds(i * 16, 16)]
                cl = jnp.minimum(jnp.maximum(raw, 0), HASH_BUCKET_SIZE - 1)
                g = fld * HASH_BUCKET_SIZE + cl
                t_v[pl.ds(i * 16, 16)] = lax.shift_right_logical(g, 3)
                o_v[pl.ds(i * 16, 16)] = lax.shift_left(
                    lax.bitwise_and(g, 7), 4)
                return c
            lax.fori_loop(0, rows_per_blk // 16, fix, 0)

            fire(g0, s0, 0)

            def pair(k, c):
                f0 = 2 * k
                f1 = f0 + 1
                pltpu.make_async_copy(
                    tbl_hbm.at[t_v.at[pl.ds(f0 * _CHUNK, _CHUNK)]],
                    g0, s0).wait()
                fire(g1, s1, f1)
                extract(g0, f0)

                @pl.when(k < (NCAT // 2) - 1)
                def _():
                    fire(g0, s0, f0 + 2)

                pltpu.make_async_copy(
                    tbl_hbm.at[t_v.at[pl.ds(f1 * _CHUNK, _CHUNK)]],
                    g1, s1).wait()
                extract(g1, f1)

                @pl.when(jnp.logical_or(k == 3, jnp.logical_or(k == 7,
                                                               k == 11)))
                def _():
                    flush(lax.div(f1, jnp.int32(8)))
                return c
            lax.fori_loop(0, NCAT // 2, pair, 0)

            pltpu.sync_copy(ridx_hbm.at[pl.ds(blk * _CHUNK, _CHUNK)], rraw_v)

            def rfix(i, c):
                raw = rraw_v[pl.ds(i * 16, 16)]
                cl = jnp.minimum(jnp.maximum(raw, 0), GEO_BUCKETS - 1)
                rt_v[pl.ds(i * 16, 16)] = lax.shift_right_logical(cl, 2)
                ro_v[pl.ds(i * 16, 16)] = lax.shift_left(
                    lax.bitwise_and(cl, 3), 5)
                return c
            lax.fori_loop(0, _CHUNK // 16, rfix, 0)

            pltpu.async_copy(rtbl_hbm.at[rt_v], g0, s0).wait()

            def rstep(p, c):
                row_vec = p * 16 + lax.iota(jnp.int32, 16)
                off_vec = ro_v[pl.ds(p * 16, 16)]
                for d in range(REGION_EMB_DIM):
                    val = plsc.load_gather(g0, [row_vec, off_vec + d])
                    colv = lax.broadcast(_CAT_COLS + d, (16,))
                    plsc.store_scatter(asm_v, [row_vec, colv], val)
                return c
            lax.fori_loop(0, _CHUNK // 16, rstep, 0)

            flush(3)
            for cb in range(4):
                pltpu.make_async_copy(
                    asm_v.at[pl.ds(0, _CHUNK), pl.ds(cb * 128, 128)],
                    x_out.at[pl.ds(blk * _CHUNK, _CHUNK),
                             pl.ds(cb * 128, 128)],
                    so).wait()
            return carry

        lax.fori_loop(0, blk_per_w, block_body, 0)

    return sc_kernel(tbl128, idx_fmajor, rtbl128, region_id)


def _gelu(x):
    return 0.5 * x * (1.0 + lax.erf(x * _SQRT_HALF))


def _tc_body(hist_ref, num_ref, xg_ref,
             w1_ref, b1_ref, w2_ref, b2_ref, w3_ref, b3_ref, out_ref):
    x = jnp.concatenate([hist_ref[...], num_ref[...],
                         xg_ref[:, :_FEAT]], axis=1)
    h = jnp.dot(x, w1_ref[...], preferred_element_type=jnp.float32) + b1_ref[...]
    h = _gelu(h)
    h = _gelu(jnp.dot(h, w2_ref[...], preferred_element_type=jnp.float32)
              + b2_ref[...])
    logits = (jnp.dot(h, w3_ref[...], preferred_element_type=jnp.float32)
              + b3_ref[...])

    bb = logits.shape[0]
    rows = lax.broadcasted_iota(jnp.int32, (K, K), 0)
    cols = lax.broadcasted_iota(jnp.int32, (K, K), 1)
    lt = jnp.where(rows <= cols, 1.0, 0.0).astype(jnp.float32)

    work = logits
    sel = jnp.zeros((bb, K), dtype=jnp.bool_)
    m1 = None
    for t in range(TOPK):
        m = jnp.max(work, axis=1, keepdims=True)
        if t == 0:
            m1 = m
        eq = work == m
        pc = jnp.dot(eq.astype(jnp.float32), lt,
                     preferred_element_type=jnp.float32)
        pick = jnp.logical_and(eq, pc == 1.0)
        sel = jnp.logical_or(sel, pick)
        work = jnp.where(pick, -jnp.inf, work)
    e = jnp.where(sel, jnp.exp(logits - m1), 0.0)
    out_ref[...] = e / jnp.sum(e, axis=1, keepdims=True)


def _tc_forward(hist_y, cur_num, xg, W1, b1, W2, b2, W3, b3, block_b=512):
    Bn = hist_y.shape[0]
    grid = (Bn // block_b,)
    row = lambda i: (i, 0)
    rep = lambda i: (0, 0)
    return pl.pallas_call(
        _tc_body,
        grid=grid,
        in_specs=[
            pl.BlockSpec((block_b, HIST), row),
            pl.BlockSpec((block_b, NUM), row),
            pl.BlockSpec((block_b, _XW), row),
            pl.BlockSpec((HIST + NUM + _FEAT, HIDDEN), rep),
            pl.BlockSpec((1, HIDDEN), rep),
            pl.BlockSpec((HIDDEN, HIDDEN), rep),
            pl.BlockSpec((1, HIDDEN), rep),
            pl.BlockSpec((HIDDEN, K), rep),
            pl.BlockSpec((1, K), rep),
        ],
        out_specs=pl.BlockSpec((block_b, K), row),
        out_shape=jax.ShapeDtypeStruct((Bn, K), jnp.float32),
    )(hist_y, cur_num, xg,
      W1, b1.reshape(1, HIDDEN),
      W2, b2.reshape(1, HIDDEN), W3, b3.reshape(1, K))


def kernel(hist_y, cur_num, cur_cat, region_id, cat_tables, region_table,
           W1, b1, W2, b2, W3, b3):
    Bn = hist_y.shape[0]
    tbl128 = cat_tables.reshape(NCAT * HASH_BUCKET_SIZE * CAT_DIM // 128, 128)
    rtbl128 = region_table.reshape(GEO_BUCKETS * REGION_EMB_DIM // 128, 128)
    idx_fmajor = (cur_cat.astype(jnp.int32)
                  .reshape(Bn // _CHUNK, _CHUNK, NCAT)
                  .transpose(0, 2, 1)
                  .reshape(Bn * NCAT))
    rid = region_id.astype(jnp.int32)
    xg = _sc_gather_call(tbl128, idx_fmajor, rtbl128, rid)
    return _tc_forward(hist_y, cur_num, xg, W1, b1, W2, b2, W3, b3)


# speedup vs baseline: 1.3062x; 1.3062x over previous
import functools

import jax
import jax.numpy as jnp
from jax import lax
from jax.experimental import pallas as pl
from jax.experimental.pallas import tpu as pltpu
from jax.experimental.pallas import tpu_sc as plsc

HASH_BUCKET_SIZE = 32768
GEO_BUCKETS = 4096
NCAT = 26
CAT_DIM = 16
REGION_EMB_DIM = 32
HIST = 200
NUM = 64
K = 64
TOPK = 8
HIDDEN = 256

_SQRT_HALF = 0.7071067811865476

_NC = 2
_NS = 16
_NW = _NC * _NS
_CHUNK = 128


def _sc_gather_call(tbl_flat, idx_fmajor, region_table, region_id):
    n_idx = idx_fmajor.shape[0]
    n_reg = region_id.shape[0]
    per_w = n_idx // _NW
    rper_w = n_reg // _NW
    rows_per_blk = _CHUNK * NCAT
    blk_per_w = per_w // rows_per_blk
    assert per_w % rows_per_blk == 0 and rper_w % _CHUNK == 0
    n_rin = rper_w // _CHUNK

    mesh = plsc.VectorSubcoreMesh(core_axis_name="c", subcore_axis_name="s")

    @functools.partial(
        pl.kernel,
        mesh=mesh,
        compiler_params=pltpu.CompilerParams(use_tc_tiling_on_sc=False),
        out_type=[
            jax.ShapeDtypeStruct((n_reg, NCAT * CAT_DIM), jnp.float32),
            jax.ShapeDtypeStruct((n_reg, REGION_EMB_DIM), jnp.float32),
        ],
        scratch_types=[
            pltpu.VMEM((per_w,), jnp.int32),
            pltpu.VMEM((rows_per_blk, CAT_DIM), jnp.float32),
            pltpu.VMEM((rper_w,), jnp.int32),
            pltpu.VMEM((rper_w, REGION_EMB_DIM), jnp.float32),
            pltpu.SemaphoreType.DMA,
            pltpu.SemaphoreType.DMA,
        ],
    )
    def sc_kernel(tbl_hbm, idx_hbm, rtbl_hbm, ridx_hbm, cat_out, reg_out,
                  idx_v, rows_v, ridx_v, rrows_v, sem_g, sem_o):
        w = lax.axis_index("s") * _NC + lax.axis_index("c")
        base = w * per_w

        pltpu.sync_copy(idx_hbm.at[pl.ds(base, per_w)], idx_v)

        def fix_cat(i, carry):
            pos = base + i * 16 + lax.iota(jnp.int32, 16)
            fld = lax.rem(lax.div(pos, jnp.int32(_CHUNK)), jnp.int32(NCAT))
            raw = idx_v[pl.ds(i * 16, 16)]
            clipped = jnp.minimum(jnp.maximum(raw, 0), HASH_BUCKET_SIZE - 1)
            idx_v[pl.ds(i * 16, 16)] = clipped + fld * HASH_BUCKET_SIZE
            return carry

        lax.fori_loop(0, per_w // 16, fix_cat, 0)

        def outer(o, carry):
            cps = []
            for c in range(NCAT):
                r0 = o * rows_per_blk + c * _CHUNK
                cps.append(pltpu.async_copy(
                    tbl_hbm.at[idx_v.at[pl.ds(r0, _CHUNK)]],
                    rows_v.at[pl.ds(c * _CHUNK, _CHUNK)], sem_g))
            for cp in cps:
                cp.wait()
            row0 = (w * blk_per_w + o) * _CHUNK
            ops = []
            for c in range(NCAT):
                ops.append(pltpu.async_copy(
                    rows_v.at[pl.ds(c * _CHUNK, _CHUNK)],
                    cat_out.at[pl.ds(row0, _CHUNK),
                               pl.ds(c * CAT_DIM, CAT_DIM)], sem_o))
            for cp in ops:
                cp.wait()
            return carry

        lax.fori_loop(0, blk_per_w, outer, 0)

        rbase = w * rper_w
        pltpu.sync_copy(ridx_hbm.at[pl.ds(rbase, rper_w)], ridx_v)

        def fix_reg(i, carry):
            raw = ridx_v[pl.ds(i * 16, 16)]
            ridx_v[pl.ds(i * 16, 16)] = jnp.minimum(
                jnp.maximum(raw, 0), GEO_BUCKETS - 1)
            return carry

        lax.fori_loop(0, rper_w // 16, fix_reg, 0)

        rcps = []
        for j in range(n_rin):
            rcps.append(pltpu.async_copy(
                rtbl_hbm.at[ridx_v.at[pl.ds(j * _CHUNK, _CHUNK)]],
                rrows_v.at[pl.ds(j * _CHUNK, _CHUNK)], sem_g))
        for cp in rcps:
            cp.wait()
        pltpu.sync_copy(rrows_v, reg_out.at[pl.ds(rbase, rper_w)])

    return sc_kernel(tbl_flat, idx_fmajor, region_table, region_id)


def _gelu(x):
    return 0.5 * x * (1.0 + lax.erf(x * _SQRT_HALF))


def _tc_body(hist_ref, num_ref, cat_ref, reg_ref,
             w1_ref, b1_ref,
             w2_ref, b2_ref, w3_ref, b3_ref, out_ref):
    x = jnp.concatenate([hist_ref[...], num_ref[...], cat_ref[...],
                         reg_ref[...]], axis=1)
    h = jnp.dot(x, w1_ref[...], preferred_element_type=jnp.float32) + b1_ref[...]
    h = _gelu(h)
    h = _gelu(jnp.dot(h, w2_ref[...], preferred_element_type=jnp.float32)
              + b2_ref[...])
    logits = (jnp.dot(h, w3_ref[...], preferred_element_type=jnp.float32)
              + b3_ref[...])

    bb = logits.shape[0]
    rows = lax.broadcasted_iota(jnp.int32, (K, K), 0)
    cols = lax.broadcasted_iota(jnp.int32, (K, K), 1)
    lt = jnp.where(rows <= cols, 1.0, 0.0).astype(jnp.float32)

    work = logits
    sel = jnp.zeros((bb, K), dtype=jnp.bool_)
    m1 = None
    for t in range(TOPK):
        m = jnp.max(work, axis=1, keepdims=True)
        if t == 0:
            m1 = m
        eq = work == m
        pc = jnp.dot(eq.astype(jnp.float32), lt,
                     preferred_element_type=jnp.float32)
        pick = jnp.logical_and(eq, pc == 1.0)
        sel = jnp.logical_or(sel, pick)
        work = jnp.where(pick, -jnp.inf, work)
    e = jnp.where(sel, jnp.exp(logits - m1), 0.0)
    out_ref[...] = e / jnp.sum(e, axis=1, keepdims=True)


def _tc_forward(hist_y, cur_num, cat_vec, reg_vec, W1, b1, W2, b2, W3, b3,
                block_b=512):
    Bn = hist_y.shape[0]
    grid = (Bn // block_b,)
    row = lambda i: (i, 0)
    rep = lambda i: (0, 0)
    return pl.pallas_call(
        _tc_body,
        grid=grid,
        in_specs=[
            pl.BlockSpec((block_b, HIST), row),
            pl.BlockSpec((block_b, NUM), row),
            pl.BlockSpec((block_b, NCAT * CAT_DIM), row),
            pl.BlockSpec((block_b, REGION_EMB_DIM), row),
            pl.BlockSpec((HIST + NUM + NCAT * CAT_DIM + REGION_EMB_DIM,
                          HIDDEN), rep),
            pl.BlockSpec((1, HIDDEN), rep),
            pl.BlockSpec((HIDDEN, HIDDEN), rep),
            pl.BlockSpec((1, HIDDEN), rep),
            pl.BlockSpec((HIDDEN, K), rep),
            pl.BlockSpec((1, K), rep),
        ],
        out_specs=pl.BlockSpec((block_b, K), row),
        out_shape=jax.ShapeDtypeStruct((Bn, K), jnp.float32),
    )(hist_y, cur_num, cat_vec, reg_vec,
      W1, b1.reshape(1, HIDDEN),
      W2, b2.reshape(1, HIDDEN), W3, b3.reshape(1, K))


def kernel(hist_y, cur_num, cur_cat, region_id, cat_tables, region_table,
           W1, b1, W2, b2, W3, b3):
    Bn = hist_y.shape[0]
    tbl_flat = cat_tables.reshape(NCAT * HASH_BUCKET_SIZE, CAT_DIM)
    idx_fmajor = (cur_cat.astype(jnp.int32)
                  .reshape(Bn // _CHUNK, _CHUNK, NCAT)
                  .transpose(0, 2, 1)
                  .reshape(Bn * NCAT))
    rid = region_id.astype(jnp.int32)
    cat_vec, reg_rows = _sc_gather_call(tbl_flat, idx_fmajor,
                                        region_table, rid)
    return _tc_forward(hist_y, cur_num, cat_vec, reg_rows,
                       W1, b1, W2, b2, W3, b3)
